# baseline (speedup 1.0000x reference)
import functools

import jax
import jax.numpy as jnp
from jax import lax
from jax.experimental import pallas as pl
from jax.experimental.pallas import tpu as pltpu
from jax.experimental.pallas import tpu_sc as plsc

_E_BLK = 2000
_N_BLK = 1000


def _stats0_body(xi_ref, xj_ref, st_ref):
    xi = xi_ref[...]
    d = xj_ref[...] - xi
    a = jnp.concatenate([xi, d], axis=1)

    @pl.when(pl.program_id(0) == 0)
    def _():
        st_ref[...] = jnp.zeros_like(st_ref)
        st_ref[2:3, :] = jnp.mean(a, 0, keepdims=True)

    c = st_ref[2:3, :]
    ac = a - c
    st_ref[0:2, :] += jnp.stack([jnp.sum(ac, 0), jnp.sum(ac * ac, 0)], 0)


def _bn_apply(h, m_ref, v_ref, g_ref, b_ref, eps=1e-5):
    return (h - m_ref[...]) / jnp.sqrt(v_ref[...] + eps) * g_ref[...] + b_ref[...]


def _acc_stats(h, st_ref):
    @pl.when(pl.program_id(0) == 0)
    def _():
        st_ref[...] = jnp.zeros_like(st_ref)
        st_ref[2:3, :] = jnp.mean(h, 0, keepdims=True)

    c = st_ref[2:3, :]
    hc = h - c
    st_ref[0:2, :] += jnp.stack([jnp.sum(hc, 0), jnp.sum(hc * hc, 0)], 0)


def _make_first_body(has_bn):
    if has_bn:
        def body(xi_ref, xj_ref, m_ref, v_ref, g_ref, be_ref, w_ref, c_ref,
                 h_ref, st_ref):
            a = jnp.concatenate([xi_ref[...], xj_ref[...] - xi_ref[...]], 1)
            a = _bn_apply(a, m_ref, v_ref, g_ref, be_ref)
            h = jnp.maximum(
                jnp.dot(a, w_ref[...], preferred_element_type=jnp.float32)
                + c_ref[...], 0.0)
            h_ref[...] = h
            _acc_stats(h, st_ref)
    else:
        def body(xi_ref, xj_ref, w_ref, c_ref, h_ref, st_ref):
            a = jnp.concatenate([xi_ref[...], xj_ref[...] - xi_ref[...]], 1)
            h = jnp.maximum(
                jnp.dot(a, w_ref[...], preferred_element_type=jnp.float32)
                + c_ref[...], 0.0)
            h_ref[...] = h
            _acc_stats(h, st_ref)
    return body


def _mm_mid_body(x_ref, m_ref, v_ref, g_ref, be_ref, w_ref, c_ref,
                 h_ref, st_ref):
    hn = _bn_apply(x_ref[...], m_ref, v_ref, g_ref, be_ref)
    h = jnp.maximum(
        jnp.dot(hn, w_ref[...], preferred_element_type=jnp.float32)
        + c_ref[...], 0.0)
    h_ref[...] = h
    _acc_stats(h, st_ref)


def _finalize_body(raw_ref, m_ref, v_ref, g_ref, be_ref, o_ref):
    val = _bn_apply(raw_ref[...], m_ref, v_ref, g_ref, be_ref)
    o_ref[...] = jnp.where(jnp.isfinite(val), val, 0.0)


def _head_body(raw_ref, m_ref, v_ref, g_ref, be_ref, w1_ref, b1_ref,
               w2_ref, b2_ref, w3_ref, b3_ref, w4_ref, b4_ref, o_ref):
    val = _bn_apply(raw_ref[...], m_ref, v_ref, g_ref, be_ref)
    h = jnp.where(jnp.isfinite(val), val, 0.0)
    h = jnp.maximum(jnp.dot(h, w1_ref[...], preferred_element_type=jnp.float32)
                    + b1_ref[...], 0.0)
    h = jnp.maximum(jnp.dot(h, w2_ref[...], preferred_element_type=jnp.float32)
                    + b2_ref[...], 0.0)
    h = jnp.dot(h, w3_ref[...], preferred_element_type=jnp.float32) + b3_ref[...]
    h = jnp.dot(h, w4_ref[...], preferred_element_type=jnp.float32) + b4_ref[...]
    o_ref[...] = 1.0 / (1.0 + jnp.exp(-h))


def _row_spec(blk, f):
    return pl.BlockSpec((blk, f), lambda i: (i, 0))


def _full_spec(shape):
    return pl.BlockSpec(shape, lambda i: (0,) * len(shape))


def _stats0(xi, xj):
    e, f = xi.shape
    return pl.pallas_call(
        _stats0_body,
        grid=(e // _E_BLK,),
        in_specs=[_row_spec(_E_BLK, f), _row_spec(_E_BLK, f)],
        out_specs=_full_spec((4, 2 * f)),
        out_shape=jax.ShapeDtypeStruct((4, 2 * f), jnp.float32),
    )(xi, xj)


def _mm_first(xi, xj, w, c, mv=None):
    e, fi = xi.shape
    fo = w.shape[1]
    row2 = [_row_spec(_E_BLK, fi), _row_spec(_E_BLK, fi)]
    if mv is not None:
        m, v, g, be = mv
        f2 = 2 * fi
        args = [xi, xj, m.reshape(1, f2), v.reshape(1, f2),
                g.reshape(1, f2), be.reshape(1, f2), w, c.reshape(1, fo)]
        in_specs = row2 + [_full_spec((1, f2))] * 4 + [
            _full_spec((f2, fo)), _full_spec((1, fo))]
    else:
        args = [xi, xj, w, c.reshape(1, fo)]
        in_specs = row2 + [_full_spec((2 * fi, fo)), _full_spec((1, fo))]
    return pl.pallas_call(
        _make_first_body(mv is not None),
        grid=(e // _E_BLK,),
        in_specs=in_specs,
        out_specs=[_row_spec(_E_BLK, fo), _full_spec((4, fo))],
        out_shape=[jax.ShapeDtypeStruct((e, fo), jnp.float32),
                   jax.ShapeDtypeStruct((4, fo), jnp.float32)],
    )(*args)


def _mm_mid(x, mv, w, c):
    e, fi = x.shape
    fo = w.shape[1]
    m, v, g, be = mv
    return pl.pallas_call(
        _mm_mid_body,
        grid=(e // _E_BLK,),
        in_specs=[_row_spec(_E_BLK, fi)] + [_full_spec((1, fi))] * 4 +
                 [_full_spec((fi, fo)), _full_spec((1, fo))],
        out_specs=[_row_spec(_E_BLK, fo), _full_spec((4, fo))],
        out_shape=[jax.ShapeDtypeStruct((e, fo), jnp.float32),
                   jax.ShapeDtypeStruct((4, fo), jnp.float32)],
    )(x, m.reshape(1, fi), v.reshape(1, fi), g.reshape(1, fi),
      be.reshape(1, fi), w, c.reshape(1, fo))


def _finalize(raw, mv):
    n, f = raw.shape
    blk = _N_BLK
    m, v, g, be = mv
    return pl.pallas_call(
        _finalize_body,
        grid=(n // blk,),
        in_specs=[_row_spec(blk, f)] + [_full_spec((1, f))] * 4,
        out_specs=_row_spec(blk, f),
        out_shape=jax.ShapeDtypeStruct((n, f), jnp.float32),
    )(raw, m.reshape(1, f), v.reshape(1, f), g.reshape(1, f),
      be.reshape(1, f))


def _head(raw, mv, p):
    n, f = raw.shape
    blk = _N_BLK
    m, v, g, be = mv
    args = [raw, m.reshape(1, f), v.reshape(1, f), g.reshape(1, f),
            be.reshape(1, f),
            p['h_w1'].T, p['h_b1'].reshape(1, -1),
            p['h_w2'].T, p['h_b2'].reshape(1, -1),
            p['h_w3'].T, p['h_b3'].reshape(1, -1),
            p['h_w4'].T, p['h_b4'].reshape(1, -1)]
    in_specs = [_row_spec(blk, f)] + [_full_spec(a.shape) for a in args[1:]]
    return pl.pallas_call(
        _head_body,
        grid=(n // blk,),
        in_specs=in_specs,
        out_specs=_row_spec(blk, 1),
        out_shape=jax.ShapeDtypeStruct((n, 1), jnp.float32),
    )(*args)


_NW = 32
_NPW = 320
_CH = 1600
_RB = 32


def _gather(table, idx):
    return jnp.take(table, idx, axis=0)


def _segmax(vals, seg, n):
    e, d = vals.shape
    nj = d // 16
    rows_total = _NW * _NPW
    mesh = plsc.VectorSubcoreMesh(core_axis_name="c", subcore_axis_name="s")
    dnums = lax.GatherDimensionNumbers(offset_dims=(), collapsed_slice_dims=(0,),
                                       start_index_map=(0,))
    nchunks = e // _CH

    @functools.partial(
        pl.kernel,
        out_type=jax.ShapeDtypeStruct((rows_total, d), jnp.float32),
        mesh=mesh,
        compiler_params=pltpu.CompilerParams(needs_layout_passes=False),
        scratch_types=[
            pltpu.VMEM((_NPW + 8, d), jnp.float32),
            pltpu.VMEM((_CH,), jnp.int32),
            pltpu.VMEM((_CH,), jnp.int32),
            pltpu.VMEM((_CH + 32,), jnp.int32),
            pltpu.VMEM((_CH + 32,), jnp.int32),
            pltpu.VMEM((16,), jnp.int32),
            pltpu.VMEM((_RB,), jnp.int32),
            pltpu.VMEM((_RB,), jnp.int32),
            pltpu.VMEM((_RB, d), jnp.float32),
            pltpu.VMEM((_RB, d), jnp.float32),
            pltpu.SemaphoreType.DMA,
            pltpu.SemaphoreType.DMA,
            pltpu.SemaphoreType.DMA,
            pltpu.SemaphoreType.DMA,
        ],
    )
    def k(vals_hbm, seg_hbm, out_hbm, nodebuf, dchA, dchB, idbuf, dlbuf,
          posbuf, ixA, ixB, rowA, rowB, csA, csB, rsA, rsB):
        wid = lax.axis_index("s") * 2 + lax.axis_index("c")
        base = wid * _NPW
        neg = jnp.full((16,), -jnp.inf, jnp.float32)
        iota = lax.iota(jnp.int32, 16)
        dch = (dchA, dchB)
        ix = (ixA, ixB)
        rowbuf = (rowA, rowB)
        cs = (csA, csB)
        rs = (rsA, rsB)

        def init_row(i, carry):
            for j in range(nj):
                nodebuf[i, pl.ds(j * 16, 16)] = neg
            return carry

        lax.fori_loop(0, _NPW + 8, init_row, 0)

        pltpu.async_copy(seg_hbm.at[pl.ds(0, _CH)], dchA, csA)

        def filt_from(dchunk, cbase):
            def filt(v, pos):
                dvec = dchunk[pl.ds(v * 32, 16)]
                m = (dvec >= base) & (dvec < base + _NPW)
                ids = cbase + v * 32 + iota
                inc = plsc.cumsum(jnp.where(m, 1, 0).astype(jnp.int32))
                plsc.store_scatter(idbuf, [pos + inc - 1], ids, mask=m)
                plsc.store_scatter(dlbuf, [pos + inc - 1], dvec - base, mask=m)
                pos = pos + plsc.all_reduce_population_count(m)
                dvec2 = dchunk[pl.ds(v * 32 + 16, 16)]
                m2 = (dvec2 >= base) & (dvec2 < base + _NPW)
                ids2 = cbase + v * 32 + 16 + iota
                inc2 = plsc.cumsum(jnp.where(m2, 1, 0).astype(jnp.int32))
                plsc.store_scatter(idbuf, [pos + inc2 - 1], ids2, mask=m2)
                plsc.store_scatter(dlbuf, [pos + inc2 - 1], dvec2 - base,
                                   mask=m2)
                return pos + plsc.all_reduce_population_count(m2)

            return lax.fori_loop(0, _CH // 32, filt,
                                 jnp.zeros((16,), jnp.int32))

        def start_rows(rb, par):
            for p in range(2):
                @pl.when(par == p)
                def _():
                    ix[p][pl.ds(0, 16)] = idbuf[pl.ds(rb, 16)]
                    ix[p][pl.ds(16, 16)] = idbuf[pl.ds(rb + 16, 16)]
                    pltpu.async_copy(vals_hbm.at[ix[p]], rowbuf[p], rs[p])

        def do_rows(rb, par):
            dl16a = dlbuf[pl.ds(rb, 16)]
            dl16b = dlbuf[pl.ds(rb + 16, 16)]
            for p in range(2):
                @pl.when(par == p)
                def _():
                    pltpu.make_async_copy(vals_hbm.at[ix[p]], rowbuf[p],
                                          rs[p]).wait()

                    def row_upd(r, rcarry):
                        rsp = jnp.broadcast_to(r, (16,))[:, None]
                        bca = lax.gather(
                            dl16a, rsp, dnums, (1,),
                            mode=lax.GatherScatterMode.PROMISE_IN_BOUNDS)
                        bcb = lax.gather(
                            dl16b, rsp, dnums, (1,),
                            mode=lax.GatherScatterMode.PROMISE_IN_BOUNDS)
                        for j in range(nj):
                            cols = iota + j * 16
                            cur = plsc.load_gather(nodebuf, [bca, cols])
                            val = rowbuf[p][r, pl.ds(j * 16, 16)]
                            plsc.store_scatter(nodebuf, [bca, cols],
                                               jnp.maximum(cur, val))
                        for j in range(nj):
                            cols = iota + j * 16
                            cur = plsc.load_gather(nodebuf, [bcb, cols])
                            val = rowbuf[p][r + 16, pl.ds(j * 16, 16)]
                            plsc.store_scatter(nodebuf, [bcb, cols],
                                               jnp.maximum(cur, val))
                        return rcarry

                    lax.fori_loop(0, 16, row_upd, 0)

        def chunk_body(c, carry):
            cpar = lax.rem(c, 2)
            cbase = c * _CH
            for p in range(2):
                @pl.when(cpar == p)
                def _():
                    pltpu.make_async_copy(seg_hbm.at[pl.ds(cbase, _CH)],
                                          dch[p], cs[p]).wait()

                    @pl.when(c + 1 < nchunks)
                    def _():
                        pltpu.async_copy(
                            seg_hbm.at[pl.ds(cbase + _CH, _CH)],
                            dch[1 - p], cs[1 - p])

                    posbuf[...] = filt_from(dch[p], cbase)

            posv = posbuf[...]
            plsc.store_scatter(dlbuf, [posv + iota],
                               jnp.full((16,), _NPW, jnp.int32))
            plsc.store_scatter(dlbuf, [posv + 16 + iota],
                               jnp.full((16,), _NPW, jnp.int32))
            plsc.store_scatter(idbuf, [posv + iota],
                               jnp.zeros((16,), jnp.int32))
            plsc.store_scatter(idbuf, [posv + 16 + iota],
                               jnp.zeros((16,), jnp.int32))
            kk = jnp.max(posv)
            nb = (kk + _RB - 1) // _RB

            @pl.when(nb > 0)
            def _():
                start_rows(0, 0)

            def batch(b, bcarry):
                par = lax.rem(b, 2)

                @pl.when(b + 1 < nb)
                def _():
                    start_rows((b + 1) * _RB, 1 - par)

                do_rows(b * _RB, par)
                return bcarry

            lax.fori_loop(0, nb, batch, 0)
            return carry

        lax.fori_loop(0, nchunks, chunk_body, 0)
        pltpu.sync_copy(nodebuf.at[pl.ds(0, _NPW)],
                        out_hbm.at[pl.ds(base, _NPW)])

    return k(vals, seg)[:n]


def _stats_mv(st, cnt):
    dm = st[0] / cnt
    return st[2] + dm, st[1] / cnt - dm * dm


def kernel(x, edge_index, params):
    p = params
    n = x.shape[0]
    e = edge_index.shape[1]
    src, dst = edge_index[0], edge_index[1]
    cnt = jnp.float32(e)

    def edge_layer(xi, xj, ws, bs, gs, betas, mv0=None):
        w1, w2, w3 = ws
        h1, st1 = _mm_first(xi, xj, w1.T, bs[0], mv0)
        m1, v1 = _stats_mv(st1, cnt)
        h2, st2 = _mm_mid(h1, (m1, v1, gs[0], betas[0]), w2.T, bs[1])
        m2, v2 = _stats_mv(st2, cnt)
        h3, st3 = _mm_mid(h2, (m2, v2, gs[1], betas[1]), w3.T, bs[2])
        m3, v3 = _stats_mv(st3, cnt)
        raw = _segmax(h3, dst, n)
        return raw, (m3, v3, gs[2], betas[2])

    xi = _gather(x, dst)
    xj = _gather(x, src)
    st0 = _stats0(xi, xj)
    m0, v0 = _stats_mv(st0, cnt)
    raw1, mv13 = edge_layer(
        xi, xj, (p['m1_w1'], p['m1_w2'], p['m1_w3']),
        (p['m1_b1'], p['m1_b2'], p['m1_b3']),
        (p['m1_bn1_g'], p['m1_bn2_g'], p['m1_bn3_g']),
        (p['m1_bn1_b'], p['m1_bn2_b'], p['m1_bn3_b']),
        (m0, v0, p['m1_bn0_g'], p['m1_bn0_b']))
    nodes1 = _finalize(raw1, mv13)

    raw2, mv23 = edge_layer(
        _gather(nodes1, dst), _gather(nodes1, src),
        (p['m2_w1'], p['m2_w2'], p['m2_w3']),
        (p['m2_b1'], p['m2_b2'], p['m2_b3']),
        (p['m2_bn1_g'], p['m2_bn2_g'], p['m2_bn3_g']),
        (p['m2_bn1_b'], p['m2_bn2_b'], p['m2_bn3_b']))

    return _head(raw2, mv23, p)


# speedup vs baseline: 1.0287x; 1.0287x over previous
import functools

import jax
import jax.numpy as jnp
from jax import lax
from jax.experimental import pallas as pl
from jax.experimental.pallas import tpu as pltpu
from jax.experimental.pallas import tpu_sc as plsc

_E_BLK = 2000
_N_BLK = 1000


def _stats0_body(xi_ref, xj_ref, st_ref):
    xi = xi_ref[...]
    d = xj_ref[...] - xi
    a = jnp.concatenate([xi, d], axis=1)

    @pl.when(pl.program_id(0) == 0)
    def _():
        st_ref[...] = jnp.zeros_like(st_ref)
        st_ref[2:3, :] = jnp.mean(a, 0, keepdims=True)

    c = st_ref[2:3, :]
    ac = a - c
    st_ref[0:2, :] += jnp.stack([jnp.sum(ac, 0), jnp.sum(ac * ac, 0)], 0)


def _bn_apply(h, m_ref, v_ref, g_ref, b_ref, eps=1e-5):
    return (h - m_ref[...]) / jnp.sqrt(v_ref[...] + eps) * g_ref[...] + b_ref[...]


def _acc_stats(h, st_ref):
    @pl.when(pl.program_id(0) == 0)
    def _():
        st_ref[...] = jnp.zeros_like(st_ref)
        st_ref[2:3, :] = jnp.mean(h, 0, keepdims=True)

    c = st_ref[2:3, :]
    hc = h - c
    st_ref[0:2, :] += jnp.stack([jnp.sum(hc, 0), jnp.sum(hc * hc, 0)], 0)


def _make_first_body(has_bn):
    if has_bn:
        def body(xi_ref, xj_ref, m_ref, v_ref, g_ref, be_ref, w_ref, c_ref,
                 h_ref, st_ref):
            a = jnp.concatenate([xi_ref[...], xj_ref[...] - xi_ref[...]], 1)
            a = _bn_apply(a, m_ref, v_ref, g_ref, be_ref)
            h = jnp.maximum(
                jnp.dot(a, w_ref[...], preferred_element_type=jnp.float32)
                + c_ref[...], 0.0)
            h_ref[...] = h
            _acc_stats(h, st_ref)
    else:
        def body(xi_ref, xj_ref, w_ref, c_ref, h_ref, st_ref):
            a = jnp.concatenate([xi_ref[...], xj_ref[...] - xi_ref[...]], 1)
            h = jnp.maximum(
                jnp.dot(a, w_ref[...], preferred_element_type=jnp.float32)
                + c_ref[...], 0.0)
            h_ref[...] = h
            _acc_stats(h, st_ref)
    return body


def _mm_mid_body(x_ref, m_ref, v_ref, g_ref, be_ref, w_ref, c_ref,
                 h_ref, st_ref):
    hn = _bn_apply(x_ref[...], m_ref, v_ref, g_ref, be_ref)
    h = jnp.maximum(
        jnp.dot(hn, w_ref[...], preferred_element_type=jnp.float32)
        + c_ref[...], 0.0)
    h_ref[...] = h
    _acc_stats(h, st_ref)


def _finalize_body(raw_ref, m_ref, v_ref, g_ref, be_ref, o_ref):
    val = _bn_apply(raw_ref[...], m_ref, v_ref, g_ref, be_ref)
    o_ref[...] = jnp.where(jnp.isfinite(val), val, 0.0)


def _head_body(raw_ref, m_ref, v_ref, g_ref, be_ref, w1_ref, b1_ref,
               w2_ref, b2_ref, w3_ref, b3_ref, w4_ref, b4_ref, o_ref):
    val = _bn_apply(raw_ref[...], m_ref, v_ref, g_ref, be_ref)
    h = jnp.where(jnp.isfinite(val), val, 0.0)
    h = jnp.maximum(jnp.dot(h, w1_ref[...], preferred_element_type=jnp.float32)
                    + b1_ref[...], 0.0)
    h = jnp.maximum(jnp.dot(h, w2_ref[...], preferred_element_type=jnp.float32)
                    + b2_ref[...], 0.0)
    h = jnp.dot(h, w3_ref[...], preferred_element_type=jnp.float32) + b3_ref[...]
    h = jnp.dot(h, w4_ref[...], preferred_element_type=jnp.float32) + b4_ref[...]
    o_ref[...] = 1.0 / (1.0 + jnp.exp(-h))


def _row_spec(blk, f):
    return pl.BlockSpec((blk, f), lambda i: (i, 0))


def _full_spec(shape):
    return pl.BlockSpec(shape, lambda i: (0,) * len(shape))


def _stats0(xi, xj):
    e, f = xi.shape
    return pl.pallas_call(
        _stats0_body,
        grid=(e // _E_BLK,),
        in_specs=[_row_spec(_E_BLK, f), _row_spec(_E_BLK, f)],
        out_specs=_full_spec((4, 2 * f)),
        out_shape=jax.ShapeDtypeStruct((4, 2 * f), jnp.float32),
    )(xi, xj)


def _mm_first(xi, xj, w, c, mv=None):
    e, fi = xi.shape
    fo = w.shape[1]
    row2 = [_row_spec(_E_BLK, fi), _row_spec(_E_BLK, fi)]
    if mv is not None:
        m, v, g, be = mv
        f2 = 2 * fi
        args = [xi, xj, m.reshape(1, f2), v.reshape(1, f2),
                g.reshape(1, f2), be.reshape(1, f2), w, c.reshape(1, fo)]
        in_specs = row2 + [_full_spec((1, f2))] * 4 + [
            _full_spec((f2, fo)), _full_spec((1, fo))]
    else:
        args = [xi, xj, w, c.reshape(1, fo)]
        in_specs = row2 + [_full_spec((2 * fi, fo)), _full_spec((1, fo))]
    return pl.pallas_call(
        _make_first_body(mv is not None),
        grid=(e // _E_BLK,),
        in_specs=in_specs,
        out_specs=[_row_spec(_E_BLK, fo), _full_spec((4, fo))],
        out_shape=[jax.ShapeDtypeStruct((e, fo), jnp.float32),
                   jax.ShapeDtypeStruct((4, fo), jnp.float32)],
    )(*args)


def _mm_mid(x, mv, w, c):
    e, fi = x.shape
    fo = w.shape[1]
    m, v, g, be = mv
    return pl.pallas_call(
        _mm_mid_body,
        grid=(e // _E_BLK,),
        in_specs=[_row_spec(_E_BLK, fi)] + [_full_spec((1, fi))] * 4 +
                 [_full_spec((fi, fo)), _full_spec((1, fo))],
        out_specs=[_row_spec(_E_BLK, fo), _full_spec((4, fo))],
        out_shape=[jax.ShapeDtypeStruct((e, fo), jnp.float32),
                   jax.ShapeDtypeStruct((4, fo), jnp.float32)],
    )(x, m.reshape(1, fi), v.reshape(1, fi), g.reshape(1, fi),
      be.reshape(1, fi), w, c.reshape(1, fo))


def _finalize(raw, mv):
    n, f = raw.shape
    blk = _N_BLK
    m, v, g, be = mv
    return pl.pallas_call(
        _finalize_body,
        grid=(n // blk,),
        in_specs=[_row_spec(blk, f)] + [_full_spec((1, f))] * 4,
        out_specs=_row_spec(blk, f),
        out_shape=jax.ShapeDtypeStruct((n, f), jnp.float32),
    )(raw, m.reshape(1, f), v.reshape(1, f), g.reshape(1, f),
      be.reshape(1, f))


def _head(raw, mv, p):
    n, f = raw.shape
    blk = _N_BLK
    m, v, g, be = mv
    args = [raw, m.reshape(1, f), v.reshape(1, f), g.reshape(1, f),
            be.reshape(1, f),
            p['h_w1'].T, p['h_b1'].reshape(1, -1),
            p['h_w2'].T, p['h_b2'].reshape(1, -1),
            p['h_w3'].T, p['h_b3'].reshape(1, -1),
            p['h_w4'].T, p['h_b4'].reshape(1, -1)]
    in_specs = [_row_spec(blk, f)] + [_full_spec(a.shape) for a in args[1:]]
    return pl.pallas_call(
        _head_body,
        grid=(n // blk,),
        in_specs=in_specs,
        out_specs=_row_spec(blk, 1),
        out_shape=jax.ShapeDtypeStruct((n, 1), jnp.float32),
    )(*args)


_NW = 32
_NPW = 320
_CH = 3200
_RB = 64


def _gather(table, idx):
    return jnp.take(table, idx, axis=0)


def _segmax(vals, seg, n):
    e, d = vals.shape
    nj = d // 16
    rows_total = _NW * _NPW
    mesh = plsc.VectorSubcoreMesh(core_axis_name="c", subcore_axis_name="s")
    dnums = lax.GatherDimensionNumbers(offset_dims=(), collapsed_slice_dims=(0,),
                                       start_index_map=(0,))
    nchunks = e // _CH

    @functools.partial(
        pl.kernel,
        out_type=jax.ShapeDtypeStruct((rows_total, d), jnp.float32),
        mesh=mesh,
        compiler_params=pltpu.CompilerParams(needs_layout_passes=False),
        scratch_types=[
            pltpu.VMEM((_NPW + 8, d), jnp.float32),
            pltpu.VMEM((_CH,), jnp.int32),
            pltpu.VMEM((_CH,), jnp.int32),
            pltpu.VMEM((_CH + 64,), jnp.int32),
            pltpu.VMEM((_CH + 64,), jnp.int32),
            pltpu.VMEM((16,), jnp.int32),
            pltpu.VMEM((_RB,), jnp.int32),
            pltpu.VMEM((_RB,), jnp.int32),
            pltpu.VMEM((_RB, d), jnp.float32),
            pltpu.VMEM((_RB, d), jnp.float32),
            pltpu.SemaphoreType.DMA,
            pltpu.SemaphoreType.DMA,
            pltpu.SemaphoreType.DMA,
            pltpu.SemaphoreType.DMA,
        ],
    )
    def k(vals_hbm, seg_hbm, out_hbm, nodebuf, dchA, dchB, idbuf, dlbuf,
          posbuf, ixA, ixB, rowA, rowB, csA, csB, rsA, rsB):
        wid = lax.axis_index("s") * 2 + lax.axis_index("c")
        base = wid * _NPW
        neg = jnp.full((16,), -jnp.inf, jnp.float32)
        iota = lax.iota(jnp.int32, 16)
        dch = (dchA, dchB)
        ix = (ixA, ixB)
        rowbuf = (rowA, rowB)
        cs = (csA, csB)
        rs = (rsA, rsB)

        def init_row(i, carry):
            for j in range(nj):
                nodebuf[i, pl.ds(j * 16, 16)] = neg
            return carry

        lax.fori_loop(0, _NPW + 8, init_row, 0)

        pltpu.async_copy(seg_hbm.at[pl.ds(0, _CH)], dchA, csA)

        def filt_from(dchunk, cbase):
            def filt(v, pos):
                dvec = dchunk[pl.ds(v * 32, 16)]
                m = (dvec >= base) & (dvec < base + _NPW)
                ids = cbase + v * 32 + iota
                inc = plsc.cumsum(jnp.where(m, 1, 0).astype(jnp.int32))
                plsc.store_scatter(idbuf, [pos + inc - 1], ids, mask=m)
                plsc.store_scatter(dlbuf, [pos + inc - 1], dvec - base, mask=m)
                pos = pos + plsc.all_reduce_population_count(m)
                dvec2 = dchunk[pl.ds(v * 32 + 16, 16)]
                m2 = (dvec2 >= base) & (dvec2 < base + _NPW)
                ids2 = cbase + v * 32 + 16 + iota
                inc2 = plsc.cumsum(jnp.where(m2, 1, 0).astype(jnp.int32))
                plsc.store_scatter(idbuf, [pos + inc2 - 1], ids2, mask=m2)
                plsc.store_scatter(dlbuf, [pos + inc2 - 1], dvec2 - base,
                                   mask=m2)
                return pos + plsc.all_reduce_population_count(m2)

            return lax.fori_loop(0, _CH // 32, filt,
                                 jnp.zeros((16,), jnp.int32))

        def start_rows(rb, par):
            for p in range(2):
                @pl.when(par == p)
                def _():
                    for q in range(_RB // 16):
                        ix[p][pl.ds(q * 16, 16)] = idbuf[pl.ds(rb + q * 16, 16)]
                    pltpu.async_copy(vals_hbm.at[ix[p]], rowbuf[p], rs[p])

        def do_rows(rb, par):
            dl16s = [dlbuf[pl.ds(rb + 16 * q, 16)] for q in range(_RB // 16)]
            for p in range(2):
                @pl.when(par == p)
                def _():
                    pltpu.make_async_copy(vals_hbm.at[ix[p]], rowbuf[p],
                                          rs[p]).wait()

                    def row_upd(r, rcarry):
                        rsp = jnp.broadcast_to(r, (16,))[:, None]
                        for q in range(_RB // 16):
                            bc = lax.gather(
                                dl16s[q], rsp, dnums, (1,),
                                mode=lax.GatherScatterMode.PROMISE_IN_BOUNDS)
                            for j in range(nj):
                                cols = iota + j * 16
                                cur = plsc.load_gather(nodebuf, [bc, cols])
                                val = rowbuf[p][r + 16 * q, pl.ds(j * 16, 16)]
                                plsc.store_scatter(nodebuf, [bc, cols],
                                                   jnp.maximum(cur, val))
                        return rcarry

                    lax.fori_loop(0, 16, row_upd, 0)

        def chunk_body(c, carry):
            cpar = lax.rem(c, 2)
            cbase = c * _CH
            for p in range(2):
                @pl.when(cpar == p)
                def _():
                    pltpu.make_async_copy(seg_hbm.at[pl.ds(cbase, _CH)],
                                          dch[p], cs[p]).wait()

                    @pl.when(c + 1 < nchunks)
                    def _():
                        pltpu.async_copy(
                            seg_hbm.at[pl.ds(cbase + _CH, _CH)],
                            dch[1 - p], cs[1 - p])

                    posbuf[...] = filt_from(dch[p], cbase)

            posv = posbuf[...]
            for q in range(_RB // 16):
                plsc.store_scatter(dlbuf, [posv + q * 16 + iota],
                                   jnp.full((16,), _NPW, jnp.int32))
                plsc.store_scatter(idbuf, [posv + q * 16 + iota],
                                   jnp.zeros((16,), jnp.int32))
            kk = jnp.max(posv)
            nb = (kk + _RB - 1) // _RB

            @pl.when(nb > 0)
            def _():
                start_rows(0, 0)

            def batch(b, bcarry):
                par = lax.rem(b, 2)

                @pl.when(b + 1 < nb)
                def _():
                    start_rows((b + 1) * _RB, 1 - par)

                do_rows(b * _RB, par)
                return bcarry

            lax.fori_loop(0, nb, batch, 0)
            return carry

        lax.fori_loop(0, nchunks, chunk_body, 0)
        pltpu.sync_copy(nodebuf.at[pl.ds(0, _NPW)],
                        out_hbm.at[pl.ds(base, _NPW)])

    return k(vals, seg)[:n]


def _stats_mv(st, cnt):
    dm = st[0] / cnt
    return st[2] + dm, st[1] / cnt - dm * dm


def kernel(x, edge_index, params):
    p = params
    n = x.shape[0]
    e = edge_index.shape[1]
    src, dst = edge_index[0], edge_index[1]
    cnt = jnp.float32(e)

    def edge_layer(xi, xj, ws, bs, gs, betas, mv0=None):
        w1, w2, w3 = ws
        h1, st1 = _mm_first(xi, xj, w1.T, bs[0], mv0)
        m1, v1 = _stats_mv(st1, cnt)
        h2, st2 = _mm_mid(h1, (m1, v1, gs[0], betas[0]), w2.T, bs[1])
        m2, v2 = _stats_mv(st2, cnt)
        h3, st3 = _mm_mid(h2, (m2, v2, gs[1], betas[1]), w3.T, bs[2])
        m3, v3 = _stats_mv(st3, cnt)
        raw = _segmax(h3, dst, n)
        return raw, (m3, v3, gs[2], betas[2])

    xi = _gather(x, dst)
    xj = _gather(x, src)
    st0 = _stats0(xi, xj)
    m0, v0 = _stats_mv(st0, cnt)
    raw1, mv13 = edge_layer(
        xi, xj, (p['m1_w1'], p['m1_w2'], p['m1_w3']),
        (p['m1_b1'], p['m1_b2'], p['m1_b3']),
        (p['m1_bn1_g'], p['m1_bn2_g'], p['m1_bn3_g']),
        (p['m1_bn1_b'], p['m1_bn2_b'], p['m1_bn3_b']),
        (m0, v0, p['m1_bn0_g'], p['m1_bn0_b']))
    nodes1 = _finalize(raw1, mv13)

    raw2, mv23 = edge_layer(
        _gather(nodes1, dst), _gather(nodes1, src),
        (p['m2_w1'], p['m2_w2'], p['m2_w3']),
        (p['m2_b1'], p['m2_b2'], p['m2_b3']),
        (p['m2_bn1_g'], p['m2_bn2_g'], p['m2_bn3_g']),
        (p['m2_bn1_b'], p['m2_bn2_b'], p['m2_bn3_b']))

    return _head(raw2, mv23, p)
